# baseline (speedup 1.0000x reference)
import jax
import jax.numpy as jnp
from jax import lax
from jax.experimental import pallas as pl
from jax.experimental.pallas import tpu as pltpu
from jax.experimental.pallas import tpu_sc as plsc

NUM_CORES = 2
NUM_SUBCORES = 16
LANES = 16

B = 16384
F = 26
V = 100000
RB = B // (NUM_CORES * NUM_SUBCORES)
NW = RB * F
IROWS = NW // 128


def _lookup_body(gx_hbm, tab_hbm, bias_hbm, out_hbm,
                 idx_v, val_v, out_v, bias_v, sem):
    c = lax.axis_index("c")
    s = lax.axis_index("s")
    t = c * NUM_SUBCORES + s

    pltpu.sync_copy(bias_hbm, bias_v)

    pltpu.sync_copy(gx_hbm.at[pl.ds(pl.multiple_of(t * NW, 8), NW)], idx_v)

    pltpu.async_copy(tab_hbm.at[idx_v], val_v, sem).wait()

    bvec = bias_v[...]

    def chunk(j, _):
        acc = bvec
        for f in range(F):
            e = lax.iota(jnp.int32, LANES) * F + (j * (LANES * F) + f)
            acc = acc + plsc.load_gather(val_v, [e])
        out_v[pl.ds(j * LANES, LANES)] = acc
        return 0
    lax.fori_loop(0, RB // LANES, chunk, 0)

    pltpu.sync_copy(out_v, out_hbm.at[pl.ds(pl.multiple_of(t * RB, 8), RB)])


@jax.jit
def _run(gx, tab, bias16):
    mesh = plsc.VectorSubcoreMesh(
        core_axis_name="c", subcore_axis_name="s",
        num_cores=NUM_CORES, num_subcores=NUM_SUBCORES)
    return pl.kernel(
        _lookup_body,
        out_type=jax.ShapeDtypeStruct((B,), jnp.float32),
        mesh=mesh,
        compiler_params=pltpu.CompilerParams(needs_layout_passes=False),
        scratch_types=[
            pltpu.VMEM((NW,), jnp.int32),
            pltpu.VMEM((NW,), jnp.float32),
            pltpu.VMEM((RB,), jnp.float32),
            pltpu.VMEM((LANES,), jnp.float32),
            pltpu.SemaphoreType.DMA,
        ],
    )(gx, tab, bias16)


def kernel(x, weights_embed, bias):
    offs = jnp.arange(F, dtype=jnp.int32) * V
    gx = (x + offs).reshape(-1)
    tab = weights_embed.reshape(-1)
    bias16 = jnp.broadcast_to(bias, (LANES,))
    out = _run(gx, tab, bias16)
    return out.reshape(B, 1)
